# baseline (speedup 1.0000x reference)
import functools

import jax
import jax.numpy as jnp
from jax import lax
from jax.experimental import pallas as pl
from jax.experimental.pallas import tpu as pltpu
from jax.experimental.pallas import tpu_sc as plsc

N = 100000
D = 128
B = 128
MARGIN = 0.0
EPS = 1e-06

NC = 2
NS = 16
NW = NC * NS
CHM = 128
DUMMY = 3 * B

CK = 2048
NBK = 34
N_TC = NBK * CK
N_SC = N - N_TC

Q8 = N_SC // 8
B8 = Q8 // NW
R8 = Q8 % NW
NSMALL = NW - R8
SPAN = 8 * (B8 + 1)
NKM = SPAN // CHM
CHT = SPAN - NKM * CHM


def _sc_pool_body(im_hbm, it_hbm, a_hbm, p_hbm, n_hbm, out_hbm,
                  acc_sh, buf, idxm, idxt, zbuf, sem0, sem1, ssem0, ssem1):
    cid = lax.axis_index("c")
    sid = lax.axis_index("s")
    wid = cid * NS + sid
    s0 = N_TC + 8 * (B8 * wid + jnp.maximum(0, wid - NSMALL))
    sems = (sem0, sem1)
    ssems = (ssem0, ssem1)

    def _z(i, _):
        zbuf[i // 8, pl.ds((i % 8) * 16, 16)] = jnp.zeros((16,), jnp.float32)
        return 0
    lax.fori_loop(0, B * 8, _z, 0)

    @pl.when(sid == 0)
    def _():
        for t in range(3):
            pltpu.sync_copy(zbuf, acc_sh.at[pl.ds(t * B, B), :])
        pltpu.sync_copy(zbuf.at[pl.ds(0, 8), :],
                        acc_sh.at[pl.ds(3 * B, 8), :])

    plsc.subcore_barrier()

    pltpu.sync_copy(im_hbm.at[wid], idxm)
    pltpu.sync_copy(it_hbm.at[wid], idxt)

    xs = (a_hbm, p_hbm, n_hbm)
    steps = [(t, k) for t in range(3) for k in range(NKM + 1)]

    def _start(c, pb):
        t, k = steps[c]
        sz = CHM if k < NKM else CHT
        row0 = s0 + CHM * k
        return pltpu.async_copy(xs[t].at[pl.ds(row0, sz), :],
                                buf.at[pb, pl.ds(0, sz), :], sems[pb])

    def _scat(c, pb):
        t, k = steps[c]
        if k < NKM:
            return pltpu.async_copy(buf.at[pb], acc_sh.at[idxm.at[t, k]],
                                    ssems[pb], add=True)
        return pltpu.async_copy(buf.at[pb, pl.ds(0, CHT), :],
                                acc_sh.at[idxt.at[t]], ssems[pb], add=True)

    nsteps = len(steps)
    copies = [None] * nsteps
    scats = [None] * nsteps
    copies[0] = _start(0, 0)
    for c in range(nsteps):
        pb = c % 2
        if c + 1 < nsteps:
            if c >= 1:
                scats[c - 1].wait()
            copies[c + 1] = _start(c + 1, (c + 1) % 2)
        copies[c].wait()
        scats[c] = _scat(c, pb)
    scats[nsteps - 2].wait()
    scats[nsteps - 1].wait()

    plsc.subcore_barrier()

    @pl.when(sid == 0)
    def _():
        pltpu.sync_copy(acc_sh.at[pl.ds(0, 3 * B), :], out_hbm.at[cid])


_sc_pool = functools.partial(
    pl.kernel,
    out_type=jax.ShapeDtypeStruct((NC, 3 * B, D), jnp.float32),
    mesh=plsc.VectorSubcoreMesh(core_axis_name="c", subcore_axis_name="s"),
    scratch_types=[
        pltpu.VMEM_SHARED((3 * B + 8, D), jnp.float32),
        pltpu.VMEM((2, CHM, D), jnp.float32),
        pltpu.VMEM((3, NKM, CHM), jnp.int32),
        pltpu.VMEM((8, CHT), jnp.int32),
        pltpu.VMEM((B, D), jnp.float32),
        pltpu.SemaphoreType.DMA,
        pltpu.SemaphoreType.DMA,
        pltpu.SemaphoreType.DMA,
        pltpu.SemaphoreType.DMA,
    ],
)(_sc_pool_body)


def _pool_tc_body(idx_ref, x_ref, out_ref):
    i = pl.program_id(0)
    idx = idx_ref[...].reshape(1, CK)
    oh = (lax.broadcasted_iota(jnp.int32, (B, CK), 0) == idx)
    y = jnp.dot(oh.astype(jnp.float32), x_ref[...],
                preferred_element_type=jnp.float32)

    @pl.when(i == 0)
    def _():
        out_ref[...] = jnp.zeros_like(out_ref)

    out_ref[...] += y


_pool_tc = pl.pallas_call(
    _pool_tc_body,
    grid=(NBK,),
    in_specs=[pl.BlockSpec((1, 1, CK), lambda i: (i, 0, 0)),
              pl.BlockSpec((CK, D), lambda i: (i, 0))],
    out_specs=pl.BlockSpec((B, D), lambda i: (0, 0)),
    out_shape=jax.ShapeDtypeStruct((B, D), jnp.float32),
)


def _loss_body(part_ref, ta_ref, tp_ref, tn_ref,
               agt_ref, pgt_ref, ngt_ref, out_ref):
    pooled = part_ref[0] + part_ref[1]
    a_p = pooled[0:B, :] + ta_ref[...]
    p_p = pooled[B:2 * B, :] + tp_ref[...]
    n_p = pooled[2 * B:3 * B, :] + tn_ref[...]
    pos_d = jnp.sqrt(jnp.sum((p_p - a_p) ** 2, axis=1, keepdims=True))
    neg_d = jnp.sqrt(jnp.sum((n_p - a_p) ** 2, axis=1, keepdims=True))
    agt = agt_ref[...]
    coeff = jnp.abs(ngt_ref[...] - agt) / (jnp.abs(pgt_ref[...] - agt) + EPS)
    loss = jnp.maximum(pos_d - coeff * neg_d + MARGIN, 0.0)
    out_ref[...] = (jnp.sum(loss) / B).reshape(1, 1)


_loss = pl.pallas_call(
    _loss_body,
    out_shape=jax.ShapeDtypeStruct((1, 1), jnp.float32),
)


def _prep_idx(ab, pb, nb):
    mains, tails, tcs = [], [], []
    dummy8 = jnp.full((NSMALL, 8), DUMMY, jnp.int32)
    lo_n = NSMALL * (SPAN - 8)
    for t, b in enumerate((ab, pb, nb)):
        raw = b.astype(jnp.int32)
        tcs.append(raw[:N_TC].reshape(NBK, 1, CK))
        arr = raw[N_TC:] + t * B
        lo = arr[:lo_n].reshape(NSMALL, SPAN - 8)
        hi = arr[lo_n:].reshape(R8, SPAN)
        main = jnp.concatenate([lo[:, :NKM * CHM], hi[:, :NKM * CHM]])
        mains.append(main.reshape(NW, NKM, CHM))
        tail_lo = jnp.concatenate([lo[:, NKM * CHM:], dummy8], axis=1)
        tails.append(jnp.concatenate([tail_lo, hi[:, NKM * CHM:]]))
    idx_main = jnp.stack(mains, axis=1)
    tail = jnp.stack(tails, axis=1)
    pad = jnp.full((NW, 5, CHT), DUMMY, jnp.int32)
    idx_tail = jnp.concatenate([tail, pad], axis=1)
    return idx_main, idx_tail, tcs


def kernel(anchor_batch, negative_batch, positive_batch, anchor, negative,
           positive, anchor_gt, negative_gt, positive_gt):
    idx_main, idx_tail, tcs = _prep_idx(anchor_batch, positive_batch,
                                        negative_batch)
    parts = _sc_pool(idx_main, idx_tail, anchor, positive, negative)
    tc_a = _pool_tc(tcs[0], anchor)
    tc_p = _pool_tc(tcs[1], positive)
    tc_n = _pool_tc(tcs[2], negative)
    out = _loss(parts, tc_a, tc_p, tc_n,
                anchor_gt.reshape(B, 1),
                positive_gt.reshape(B, 1),
                negative_gt.reshape(B, 1))
    return out[0, 0]


# speedup vs baseline: 1.2018x; 1.2018x over previous
import functools

import jax
import jax.numpy as jnp
from jax import lax
from jax.experimental import pallas as pl
from jax.experimental.pallas import tpu as pltpu
from jax.experimental.pallas import tpu_sc as plsc

N = 100000
D = 128
B = 128
MARGIN = 0.0
EPS = 1e-06

NC = 2
NS = 16
NW = NC * NS
CHM = 128
DUMMY = 3 * B

CK = 2048
NBK = 24
N_TC = NBK * CK
N_SC = N - N_TC

Q8 = N_SC // 8
B8 = Q8 // NW
R8 = Q8 % NW
NSMALL = NW - R8
SPAN = 8 * (B8 + 1)
NKM = SPAN // CHM
CHT = SPAN - NKM * CHM


def _sc_pool_body(im_hbm, it_hbm, a_hbm, p_hbm, n_hbm, out_hbm,
                  acc_sh, buf, idxm, idxt, zbuf, sem0, sem1, ssem0, ssem1):
    cid = lax.axis_index("c")
    sid = lax.axis_index("s")
    wid = cid * NS + sid
    s0 = N_TC + 8 * (B8 * wid + jnp.maximum(0, wid - NSMALL))
    sems = (sem0, sem1)
    ssems = (ssem0, ssem1)

    def _z(i, _):
        zbuf[i // 8, pl.ds((i % 8) * 16, 16)] = jnp.zeros((16,), jnp.float32)
        return 0
    lax.fori_loop(0, B * 8, _z, 0)

    @pl.when(sid == 0)
    def _():
        for t in range(3):
            pltpu.sync_copy(zbuf, acc_sh.at[pl.ds(t * B, B), :])
        pltpu.sync_copy(zbuf.at[pl.ds(0, 8), :],
                        acc_sh.at[pl.ds(3 * B, 8), :])

    plsc.subcore_barrier()

    pltpu.sync_copy(im_hbm.at[wid], idxm)
    pltpu.sync_copy(it_hbm.at[wid], idxt)

    xs = (a_hbm, p_hbm, n_hbm)
    steps = [(t, k) for t in range(3) for k in range(NKM + 1)]

    def _start(c, pb):
        t, k = steps[c]
        sz = CHM if k < NKM else CHT
        row0 = s0 + CHM * k
        return pltpu.async_copy(xs[t].at[pl.ds(row0, sz), :],
                                buf.at[pb, pl.ds(0, sz), :], sems[pb])

    def _scat(c, pb):
        t, k = steps[c]
        if k < NKM:
            return pltpu.async_copy(buf.at[pb], acc_sh.at[idxm.at[t, k]],
                                    ssems[pb], add=True)
        return pltpu.async_copy(buf.at[pb, pl.ds(0, CHT), :],
                                acc_sh.at[idxt.at[t]], ssems[pb], add=True)

    nsteps = len(steps)
    copies = [None] * nsteps
    scats = [None] * nsteps
    copies[0] = _start(0, 0)
    for c in range(nsteps):
        pb = c % 2
        if c + 1 < nsteps:
            if c >= 1:
                scats[c - 1].wait()
            copies[c + 1] = _start(c + 1, (c + 1) % 2)
        copies[c].wait()
        scats[c] = _scat(c, pb)
    scats[nsteps - 2].wait()
    scats[nsteps - 1].wait()

    plsc.subcore_barrier()

    @pl.when(sid == 0)
    def _():
        pltpu.sync_copy(acc_sh.at[pl.ds(0, 3 * B), :], out_hbm.at[cid])


_sc_pool = functools.partial(
    pl.kernel,
    out_type=jax.ShapeDtypeStruct((NC, 3 * B, D), jnp.float32),
    mesh=plsc.VectorSubcoreMesh(core_axis_name="c", subcore_axis_name="s"),
    scratch_types=[
        pltpu.VMEM_SHARED((3 * B + 8, D), jnp.float32),
        pltpu.VMEM((2, CHM, D), jnp.float32),
        pltpu.VMEM((3, NKM, CHM), jnp.int32),
        pltpu.VMEM((8, CHT), jnp.int32),
        pltpu.VMEM((B, D), jnp.float32),
        pltpu.SemaphoreType.DMA,
        pltpu.SemaphoreType.DMA,
        pltpu.SemaphoreType.DMA,
        pltpu.SemaphoreType.DMA,
    ],
)(_sc_pool_body)


def _pool_tc_body(idx_ref, x_ref, out_ref):
    i = pl.program_id(0)
    idx = idx_ref[...].reshape(1, CK)
    oh = (lax.broadcasted_iota(jnp.int32, (B, CK), 0) == idx)
    y = jnp.dot(oh.astype(jnp.float32), x_ref[...],
                preferred_element_type=jnp.float32)

    @pl.when(i == 0)
    def _():
        out_ref[...] = jnp.zeros_like(out_ref)

    out_ref[...] += y


_pool_tc = pl.pallas_call(
    _pool_tc_body,
    grid=(NBK,),
    in_specs=[pl.BlockSpec((1, 1, CK), lambda i: (i, 0, 0)),
              pl.BlockSpec((CK, D), lambda i: (i, 0))],
    out_specs=pl.BlockSpec((B, D), lambda i: (0, 0)),
    out_shape=jax.ShapeDtypeStruct((B, D), jnp.float32),
)


def _loss_body(part_ref, ta_ref, tp_ref, tn_ref,
               agt_ref, pgt_ref, ngt_ref, out_ref):
    pooled = part_ref[0] + part_ref[1]
    a_p = pooled[0:B, :] + ta_ref[...]
    p_p = pooled[B:2 * B, :] + tp_ref[...]
    n_p = pooled[2 * B:3 * B, :] + tn_ref[...]
    pos_d = jnp.sqrt(jnp.sum((p_p - a_p) ** 2, axis=1, keepdims=True))
    neg_d = jnp.sqrt(jnp.sum((n_p - a_p) ** 2, axis=1, keepdims=True))
    agt = agt_ref[...]
    coeff = jnp.abs(ngt_ref[...] - agt) / (jnp.abs(pgt_ref[...] - agt) + EPS)
    loss = jnp.maximum(pos_d - coeff * neg_d + MARGIN, 0.0)
    out_ref[...] = (jnp.sum(loss) / B).reshape(1, 1)


_loss = pl.pallas_call(
    _loss_body,
    out_shape=jax.ShapeDtypeStruct((1, 1), jnp.float32),
)


def _prep_idx(ab, pb, nb):
    mains, tails, tcs = [], [], []
    dummy8 = jnp.full((NSMALL, 8), DUMMY, jnp.int32)
    lo_n = NSMALL * (SPAN - 8)
    for t, b in enumerate((ab, pb, nb)):
        raw = b.astype(jnp.int32)
        tcs.append(raw[:N_TC].reshape(NBK, 1, CK))
        arr = raw[N_TC:] + t * B
        lo = arr[:lo_n].reshape(NSMALL, SPAN - 8)
        hi = arr[lo_n:].reshape(R8, SPAN)
        main = jnp.concatenate([lo[:, :NKM * CHM], hi[:, :NKM * CHM]])
        mains.append(main.reshape(NW, NKM, CHM))
        tail_lo = jnp.concatenate([lo[:, NKM * CHM:], dummy8], axis=1)
        tails.append(jnp.concatenate([tail_lo, hi[:, NKM * CHM:]]))
    idx_main = jnp.stack(mains, axis=1)
    tail = jnp.stack(tails, axis=1)
    pad = jnp.full((NW, 5, CHT), DUMMY, jnp.int32)
    idx_tail = jnp.concatenate([tail, pad], axis=1)
    return idx_main, idx_tail, tcs


def kernel(anchor_batch, negative_batch, positive_batch, anchor, negative,
           positive, anchor_gt, negative_gt, positive_gt):
    idx_main, idx_tail, tcs = _prep_idx(anchor_batch, positive_batch,
                                        negative_batch)
    parts = _sc_pool(idx_main, idx_tail, anchor, positive, negative)
    tc_a = _pool_tc(tcs[0], anchor)
    tc_p = _pool_tc(tcs[1], positive)
    tc_n = _pool_tc(tcs[2], negative)
    out = _loss(parts, tc_a, tc_p, tc_n,
                anchor_gt.reshape(B, 1),
                positive_gt.reshape(B, 1),
                negative_gt.reshape(B, 1))
    return out[0, 0]


# speedup vs baseline: 1.3146x; 1.0939x over previous
import functools

import jax
import jax.numpy as jnp
from jax import lax
from jax.experimental import pallas as pl
from jax.experimental.pallas import tpu as pltpu
from jax.experimental.pallas import tpu_sc as plsc

N = 100000
D = 128
B = 128
MARGIN = 0.0
EPS = 1e-06

NC = 2
NS = 16
NW = NC * NS
CHM = 128
DUMMY = 3 * B

CK = 2048
NBK = 20
N_TC = NBK * CK
N_SC = N - N_TC

Q8 = N_SC // 8
B8 = Q8 // NW
R8 = Q8 % NW
NSMALL = NW - R8
SPAN = 8 * (B8 + 1)
NKM = SPAN // CHM
CHT = SPAN - NKM * CHM


def _sc_pool_body(im_hbm, it_hbm, a_hbm, p_hbm, n_hbm, out_hbm,
                  acc_sh, buf, idxm, idxt, zbuf, sem0, sem1, ssem0, ssem1):
    cid = lax.axis_index("c")
    sid = lax.axis_index("s")
    wid = cid * NS + sid
    s0 = N_TC + 8 * (B8 * wid + jnp.maximum(0, wid - NSMALL))
    sems = (sem0, sem1)
    ssems = (ssem0, ssem1)

    def _z(i, _):
        zbuf[i // 8, pl.ds((i % 8) * 16, 16)] = jnp.zeros((16,), jnp.float32)
        return 0
    lax.fori_loop(0, B * 8, _z, 0)

    @pl.when(sid == 0)
    def _():
        for t in range(3):
            pltpu.sync_copy(zbuf, acc_sh.at[pl.ds(t * B, B), :])
        pltpu.sync_copy(zbuf.at[pl.ds(0, 8), :],
                        acc_sh.at[pl.ds(3 * B, 8), :])

    plsc.subcore_barrier()

    pltpu.sync_copy(im_hbm.at[wid], idxm)
    pltpu.sync_copy(it_hbm.at[wid], idxt)

    xs = (a_hbm, p_hbm, n_hbm)
    steps = [(t, k) for t in range(3) for k in range(NKM + 1)]

    def _start(c, pb):
        t, k = steps[c]
        sz = CHM if k < NKM else CHT
        row0 = s0 + CHM * k
        return pltpu.async_copy(xs[t].at[pl.ds(row0, sz), :],
                                buf.at[pb, pl.ds(0, sz), :], sems[pb])

    def _scat(c, pb):
        t, k = steps[c]
        if k < NKM:
            return pltpu.async_copy(buf.at[pb], acc_sh.at[idxm.at[t, k]],
                                    ssems[pb], add=True)
        return pltpu.async_copy(buf.at[pb, pl.ds(0, CHT), :],
                                acc_sh.at[idxt.at[t]], ssems[pb], add=True)

    nsteps = len(steps)
    copies = [None] * nsteps
    scats = [None] * nsteps
    copies[0] = _start(0, 0)
    for c in range(nsteps):
        pb = c % 2
        if c + 1 < nsteps:
            if c >= 1:
                scats[c - 1].wait()
            copies[c + 1] = _start(c + 1, (c + 1) % 2)
        copies[c].wait()
        scats[c] = _scat(c, pb)
    scats[nsteps - 2].wait()
    scats[nsteps - 1].wait()

    plsc.subcore_barrier()

    @pl.when(sid == 0)
    def _():
        pltpu.sync_copy(acc_sh.at[pl.ds(0, 3 * B), :], out_hbm.at[cid])


_sc_pool = functools.partial(
    pl.kernel,
    out_type=jax.ShapeDtypeStruct((NC, 3 * B, D), jnp.float32),
    mesh=plsc.VectorSubcoreMesh(core_axis_name="c", subcore_axis_name="s"),
    scratch_types=[
        pltpu.VMEM_SHARED((3 * B + 8, D), jnp.float32),
        pltpu.VMEM((2, CHM, D), jnp.float32),
        pltpu.VMEM((3, NKM, CHM), jnp.int32),
        pltpu.VMEM((8, CHT), jnp.int32),
        pltpu.VMEM((B, D), jnp.float32),
        pltpu.SemaphoreType.DMA,
        pltpu.SemaphoreType.DMA,
        pltpu.SemaphoreType.DMA,
        pltpu.SemaphoreType.DMA,
    ],
)(_sc_pool_body)


def _pool_tc_body(idx_ref, x_ref, out_ref):
    i = pl.program_id(0)
    idx = idx_ref[...].reshape(1, CK)
    oh = (lax.broadcasted_iota(jnp.int32, (B, CK), 0) == idx)
    y = jnp.dot(oh.astype(jnp.float32), x_ref[...],
                preferred_element_type=jnp.float32)

    @pl.when(i == 0)
    def _():
        out_ref[...] = jnp.zeros_like(out_ref)

    out_ref[...] += y


_pool_tc = pl.pallas_call(
    _pool_tc_body,
    grid=(NBK,),
    in_specs=[pl.BlockSpec((1, 1, CK), lambda i: (i, 0, 0)),
              pl.BlockSpec((CK, D), lambda i: (i, 0))],
    out_specs=pl.BlockSpec((B, D), lambda i: (0, 0)),
    out_shape=jax.ShapeDtypeStruct((B, D), jnp.float32),
)


def _loss_body(part_ref, ta_ref, tp_ref, tn_ref,
               agt_ref, pgt_ref, ngt_ref, out_ref):
    pooled = part_ref[0] + part_ref[1]
    a_p = pooled[0:B, :] + ta_ref[...]
    p_p = pooled[B:2 * B, :] + tp_ref[...]
    n_p = pooled[2 * B:3 * B, :] + tn_ref[...]
    pos_d = jnp.sqrt(jnp.sum((p_p - a_p) ** 2, axis=1, keepdims=True))
    neg_d = jnp.sqrt(jnp.sum((n_p - a_p) ** 2, axis=1, keepdims=True))
    agt = agt_ref[...]
    coeff = jnp.abs(ngt_ref[...] - agt) / (jnp.abs(pgt_ref[...] - agt) + EPS)
    loss = jnp.maximum(pos_d - coeff * neg_d + MARGIN, 0.0)
    out_ref[...] = (jnp.sum(loss) / B).reshape(1, 1)


_loss = pl.pallas_call(
    _loss_body,
    out_shape=jax.ShapeDtypeStruct((1, 1), jnp.float32),
)


def _prep_idx(ab, pb, nb):
    mains, tails, tcs = [], [], []
    dummy8 = jnp.full((NSMALL, 8), DUMMY, jnp.int32)
    lo_n = NSMALL * (SPAN - 8)
    for t, b in enumerate((ab, pb, nb)):
        raw = b.astype(jnp.int32)
        tcs.append(raw[:N_TC].reshape(NBK, 1, CK))
        arr = raw[N_TC:] + t * B
        lo = arr[:lo_n].reshape(NSMALL, SPAN - 8)
        hi = arr[lo_n:].reshape(R8, SPAN)
        main = jnp.concatenate([lo[:, :NKM * CHM], hi[:, :NKM * CHM]])
        mains.append(main.reshape(NW, NKM, CHM))
        tail_lo = jnp.concatenate([lo[:, NKM * CHM:], dummy8], axis=1)
        tails.append(jnp.concatenate([tail_lo, hi[:, NKM * CHM:]]))
    idx_main = jnp.stack(mains, axis=1)
    tail = jnp.stack(tails, axis=1)
    pad = jnp.full((NW, 5, CHT), DUMMY, jnp.int32)
    idx_tail = jnp.concatenate([tail, pad], axis=1)
    return idx_main, idx_tail, tcs


def kernel(anchor_batch, negative_batch, positive_batch, anchor, negative,
           positive, anchor_gt, negative_gt, positive_gt):
    idx_main, idx_tail, tcs = _prep_idx(anchor_batch, positive_batch,
                                        negative_batch)
    parts = _sc_pool(idx_main, idx_tail, anchor, positive, negative)
    tc_a = _pool_tc(tcs[0], anchor)
    tc_p = _pool_tc(tcs[1], positive)
    tc_n = _pool_tc(tcs[2], negative)
    out = _loss(parts, tc_a, tc_p, tc_n,
                anchor_gt.reshape(B, 1),
                positive_gt.reshape(B, 1),
                negative_gt.reshape(B, 1))
    return out[0, 0]
